# Initial kernel scaffold; baseline (speedup 1.0000x reference)
import jax
import jax.numpy as jnp
from jax.experimental import pallas as pl


def kernel(x, edge_index, known_mask):
    raise NotImplementedError("write your pallas kernel here")


# speedup vs baseline: 13.2400x; 13.2400x over previous
import functools

import jax
import jax.numpy as jnp
from jax import lax
from jax.experimental import pallas as pl
from jax.experimental.pallas import tpu as pltpu
from jax.experimental.pallas import tpu_sc as plsc

N = 10000
E = 320000
D = 128
K = 5000

NC = 2
NS = 16
NW = NC * NS
L = 16

NPAD = 10240
RPW = NPAD // NW
EPW = E // NW
KPAD = 5120
KPW = KPAD // NW

ECH = 80
NCH = EPW // ECH

RSUB = 80
RPS = NPAD // NS
BSUB = 64


def _wid():
    return lax.axis_index("s") * NC + lax.axis_index("c")


def _rsqrt16(t16):
    u = plsc.bitcast(t16, jnp.int32)
    g = plsc.bitcast(jnp.int32(0x5F3759DF) - lax.shift_right_logical(u, 1),
                     jnp.float32)
    for _ in range(3):
        g = g * (1.5 - 0.5 * t16 * g * g)
    return g


def _k1_body(row_hbm, col_hbm, km_hbm, part_hbm,
             rbuf, cbuf, kbuf, deg_l, kf_l, m_l):
    w = _wid()
    ones = jnp.ones((L,), jnp.float32)

    def zero_body(g, _):
        z = jnp.zeros((L,), jnp.float32)
        deg_l[pl.ds(g * L, L)] = z
        kf_l[pl.ds(g * L, L)] = z
        m_l[pl.ds(g * L, L)] = z
        return _

    lax.fori_loop(0, NPAD // L, zero_body, None)

    pltpu.sync_copy(row_hbm.at[pl.ds(w * EPW, EPW)], rbuf)
    pltpu.sync_copy(col_hbm.at[pl.ds(w * EPW, EPW)], cbuf)
    pltpu.sync_copy(km_hbm.at[pl.ds(w * KPW, KPW)], kbuf)

    def edge_body(g, _):
        r16 = rbuf[pl.ds(g * L, L)]
        c16 = cbuf[pl.ds(g * L, L)]
        plsc.addupdate_scatter(deg_l, [r16], ones)
        self_v = jnp.where(r16 == c16, 1.0, 0.0).astype(jnp.float32)
        plsc.addupdate_scatter(m_l, [r16], self_v)
        return _

    lax.fori_loop(0, EPW // L, edge_body, None)

    def km_body(g, _):
        k16 = kbuf[pl.ds(g * L, L)]
        plsc.addupdate_scatter(kf_l, [k16], ones)
        return _

    lax.fori_loop(0, KPW // L, km_body, None)

    pltpu.sync_copy(deg_l, part_hbm.at[pl.ds((w * 3 + 0) * NPAD, NPAD)])
    pltpu.sync_copy(kf_l, part_hbm.at[pl.ds((w * 3 + 1) * NPAD, NPAD)])
    pltpu.sync_copy(m_l, part_hbm.at[pl.ds((w * 3 + 2) * NPAD, NPAD)])


def _k2_body(part_hbm, x_hbm, dis_hbm, kn_hbm, m_hbm, y_hbm, ys_hbm,
             pbuf, dis_l, kn_l, m_acc, xbuf, ybuf, ysbuf):
    w = _wid()
    base = w * RPW
    for t in range(NW):
        for ch in range(3):
            pltpu.sync_copy(
                part_hbm.at[pl.ds((t * 3 + ch) * NPAD + base, RPW)],
                pbuf.at[pl.ds((t * 3 + ch) * RPW, RPW)])

    def red_body(g, _):
        deg16 = jnp.zeros((L,), jnp.float32)
        kf16 = jnp.zeros((L,), jnp.float32)
        m16 = jnp.zeros((L,), jnp.float32)
        for t in range(NW):
            deg16 = deg16 + pbuf[pl.ds((t * 3 + 0) * RPW + g * L, L)]
            kf16 = kf16 + pbuf[pl.ds((t * 3 + 1) * RPW + g * L, L)]
            m16 = m16 + pbuf[pl.ds((t * 3 + 2) * RPW + g * L, L)]
        pos = jnp.where(deg16 > 0.0, 1.0, 0.0).astype(jnp.float32)
        dis16 = _rsqrt16(jnp.maximum(deg16, 1.0)) * pos
        kn16 = jnp.where(kf16 > 0.0, 1.0, 0.0).astype(jnp.float32)
        dis_l[pl.ds(g * L, L)] = dis16
        kn_l[pl.ds(g * L, L)] = kn16
        m_acc[pl.ds(g * L, L)] = m16
        return _

    lax.fori_loop(0, RPW // L, red_body, None)

    pltpu.sync_copy(dis_l, dis_hbm.at[pl.ds(base, RPW)])
    pltpu.sync_copy(kn_l, kn_hbm.at[pl.ds(base, RPW)])
    pltpu.sync_copy(m_acc, m_hbm.at[pl.ds(base, RPW)])

    for b in range(RPW // RSUB):
        rb = base + b * RSUB
        pltpu.sync_copy(x_hbm.at[pl.ds(rb, RSUB), :], xbuf)

        def grp_body(g, _):
            kvec = kn_l[pl.ds(b * RSUB + g * L, L)]
            dvec = dis_l[pl.ds(b * RSUB + g * L, L)]
            for r in range(L):
                row = g * L + r
                s16 = jnp.zeros((L,), jnp.float32)
                xv = []
                for j in range(D // L):
                    v = xbuf[row, pl.ds(j * L, L)]
                    xv.append(v)
                    s16 = s16 + v * v
                tot = jnp.sum(s16)
                t16 = jnp.full((L,), tot, jnp.float32)
                inv16 = _rsqrt16(jnp.maximum(t16, 1e-24))
                kn16 = jnp.full((L,), kvec[r], jnp.float32)
                ds16 = jnp.full((L,), dvec[r], jnp.float32)
                for j in range(D // L):
                    yv = xv[j] * inv16 * kn16
                    ybuf[row, pl.ds(j * L, L)] = yv
                    ysbuf[row, pl.ds(j * L, L)] = yv * ds16
            return _

        lax.fori_loop(0, RSUB // L, grp_body, None)
        pltpu.sync_copy(ybuf, y_hbm.at[pl.ds(rb, RSUB), :])
        pltpu.sync_copy(ysbuf, ys_hbm.at[pl.ds(rb, RSUB), :])


def _k3_body(row_hbm, col_hbm, ys_hbm, part_hbm,
             acc, ridx, cidx, gbuf, zbuf, sem):
    c = lax.axis_index("c")
    s = lax.axis_index("s")
    w = s * NC + c

    def zero_rows(r, _):
        for j in range(D // L):
            zbuf[r, pl.ds(j * L, L)] = jnp.zeros((L,), jnp.float32)
        return _

    lax.fori_loop(0, 128, zero_rows, None)
    for kk in range(RPS // 128):
        pltpu.sync_copy(zbuf, acc.at[pl.ds(s * RPS + kk * 128, 128), :])
    plsc.subcore_barrier()

    ebase = w * EPW

    def chunk_body(g, _):
        off = ebase + g * ECH
        pltpu.sync_copy(row_hbm.at[pl.ds(off, ECH)], ridx)
        pltpu.sync_copy(col_hbm.at[pl.ds(off, ECH)], cidx)
        pltpu.async_copy(ys_hbm.at[cidx], gbuf, sem).wait()
        pltpu.sync_copy(gbuf, acc.at[ridx], add=True)
        return _

    lax.fori_loop(0, NCH, chunk_body, None)
    plsc.subcore_barrier()

    for kk in range(RPS // 128):
        rb = s * RPS + kk * 128
        pltpu.sync_copy(acc.at[pl.ds(rb, 128), :], zbuf)
        pltpu.sync_copy(zbuf, part_hbm.at[c, pl.ds(rb, 128), :])


def _k4_body(part_hbm, dis_hbm, kn_hbm, m_hbm, y_hbm, ys_hbm, out_hbm,
             p0, p1, yb, ysb, ob, dis_l, kn_l, m_l):
    w = _wid()
    base = w * RPW
    pltpu.sync_copy(dis_hbm.at[pl.ds(base, RPW)], dis_l)
    pltpu.sync_copy(kn_hbm.at[pl.ds(base, RPW)], kn_l)
    pltpu.sync_copy(m_hbm.at[pl.ds(base, RPW)], m_l)

    for b in range(RPW // BSUB):
        rb = base + b * BSUB
        pltpu.sync_copy(part_hbm.at[0, pl.ds(rb, BSUB), :], p0)
        pltpu.sync_copy(part_hbm.at[1, pl.ds(rb, BSUB), :], p1)
        pltpu.sync_copy(y_hbm.at[pl.ds(rb, BSUB), :], yb)
        pltpu.sync_copy(ys_hbm.at[pl.ds(rb, BSUB), :], ysb)

        def grp_body(g, _):
            dvec = dis_l[pl.ds(b * BSUB + g * L, L)]
            kvec = kn_l[pl.ds(b * BSUB + g * L, L)]
            mvec = m_l[pl.ds(b * BSUB + g * L, L)]
            for r in range(L):
                row = g * L + r
                d16 = jnp.full((L,), dvec[r], jnp.float32)
                k16 = jnp.full((L,), kvec[r], jnp.float32)
                m16 = jnp.full((L,), mvec[r], jnp.float32)
                nk16 = 1.0 - k16
                for j in range(D // L):
                    sl = pl.ds(j * L, L)
                    prop = d16 * (p0[row, sl] + p1[row, sl]
                                  - m16 * ysb[row, sl])
                    ob[row, sl] = k16 * yb[row, sl] + nk16 * prop
            return _

        lax.fori_loop(0, BSUB // L, grp_body, None)
        pltpu.sync_copy(ob, out_hbm.at[pl.ds(rb, BSUB), :])


@functools.cache
def _build():
    mesh = plsc.VectorSubcoreMesh(
        core_axis_name="c", subcore_axis_name="s",
        num_cores=NC, num_subcores=NS)
    f32 = jnp.float32
    cparams = pltpu.CompilerParams(needs_layout_passes=False)

    k1 = pl.kernel(
        _k1_body,
        out_type=jax.ShapeDtypeStruct((NW * 3 * NPAD,), f32),
        mesh=mesh,
        compiler_params=cparams,
        scratch_types=[
            pltpu.VMEM((EPW,), jnp.int32),
            pltpu.VMEM((EPW,), jnp.int32),
            pltpu.VMEM((KPW,), jnp.int32),
            pltpu.VMEM((NPAD,), f32),
            pltpu.VMEM((NPAD,), f32),
            pltpu.VMEM((NPAD,), f32),
        ],
    )
    k2 = pl.kernel(
        _k2_body,
        out_type=[
            jax.ShapeDtypeStruct((NPAD,), f32),
            jax.ShapeDtypeStruct((NPAD,), f32),
            jax.ShapeDtypeStruct((NPAD,), f32),
            jax.ShapeDtypeStruct((NPAD, D), f32),
            jax.ShapeDtypeStruct((NPAD, D), f32),
        ],
        mesh=mesh,
        compiler_params=cparams,
        scratch_types=[
            pltpu.VMEM((NW * 3 * RPW,), f32),
            pltpu.VMEM((RPW,), f32),
            pltpu.VMEM((RPW,), f32),
            pltpu.VMEM((RPW,), f32),
            pltpu.VMEM((RSUB, D), f32),
            pltpu.VMEM((RSUB, D), f32),
            pltpu.VMEM((RSUB, D), f32),
        ],
    )
    k3 = pl.kernel(
        _k3_body,
        out_type=jax.ShapeDtypeStruct((NC, NPAD, D), f32),
        mesh=mesh,
        compiler_params=cparams,
        scratch_types=[
            pltpu.VMEM_SHARED((NPAD, D), f32),
            pltpu.VMEM((ECH,), jnp.int32),
            pltpu.VMEM((ECH,), jnp.int32),
            pltpu.VMEM((ECH, D), f32),
            pltpu.VMEM((128, D), f32),
            pltpu.SemaphoreType.DMA,
        ],
    )
    k4 = pl.kernel(
        _k4_body,
        out_type=jax.ShapeDtypeStruct((NPAD, D), f32),
        mesh=mesh,
        compiler_params=cparams,
        scratch_types=[
            pltpu.VMEM((BSUB, D), f32),
            pltpu.VMEM((BSUB, D), f32),
            pltpu.VMEM((BSUB, D), f32),
            pltpu.VMEM((BSUB, D), f32),
            pltpu.VMEM((BSUB, D), f32),
            pltpu.VMEM((RPW,), f32),
            pltpu.VMEM((RPW,), f32),
            pltpu.VMEM((RPW,), f32),
        ],
    )
    return k1, k2, k3, k4


@jax.jit
def kernel(x, edge_index, known_mask):
    k1, k2, k3, k4 = _build()
    row = edge_index[0]
    col = edge_index[1]
    x_pad = jnp.concatenate(
        [x, jnp.zeros((NPAD - N, D), jnp.float32)], axis=0)
    km_pad = jnp.concatenate(
        [known_mask, jnp.full((KPAD - K,), N, jnp.int32)], axis=0)

    part = k1(row, col, km_pad)
    dis, kn, m, y, ys = k2(part, x_pad)
    pacc = k3(row, col, ys)
    out_pad = k4(pacc, dis, kn, m, y, ys)
    return out_pad[:N]


# speedup vs baseline: 21.7571x; 1.6433x over previous
import functools

import jax
import jax.numpy as jnp
from jax import lax
from jax.experimental import pallas as pl
from jax.experimental.pallas import tpu as pltpu
from jax.experimental.pallas import tpu_sc as plsc

N = 10000
E = 320000
D = 128
K = 5000

NC = 2
NS = 16
NW = NC * NS
L = 16

NPAD = 10240
RPW = NPAD // NW
EPW = E // NW
KPAD = 5120
KPW = KPAD // NW

ECH = 80
NCH = EPW // ECH

RSUB = 80
RPS = NPAD // NS
BSUB = 64


def _wid():
    return lax.axis_index("s") * NC + lax.axis_index("c")


def _rsqrt16(t16):
    u = plsc.bitcast(t16, jnp.int32)
    g = plsc.bitcast(jnp.int32(0x5F3759DF) - lax.shift_right_logical(u, 1),
                     jnp.float32)
    for _ in range(3):
        g = g * (1.5 - 0.5 * t16 * g * g)
    return g


def _k1_body(row_hbm, col_hbm, km_hbm, part_hbm,
             rbuf, cbuf, kbuf, deg_l, kf_l, m_l):
    w = _wid()
    ones = jnp.ones((L,), jnp.float32)

    def zero_body(g, _):
        z = jnp.zeros((L,), jnp.float32)
        deg_l[pl.ds(g * L, L)] = z
        kf_l[pl.ds(g * L, L)] = z
        m_l[pl.ds(g * L, L)] = z
        return _

    lax.fori_loop(0, NPAD // L, zero_body, None)

    pltpu.sync_copy(row_hbm.at[pl.ds(w * EPW, EPW)], rbuf)
    pltpu.sync_copy(col_hbm.at[pl.ds(w * EPW, EPW)], cbuf)
    pltpu.sync_copy(km_hbm.at[pl.ds(w * KPW, KPW)], kbuf)

    def edge_body(g, _):
        r16 = rbuf[pl.ds(g * L, L)]
        c16 = cbuf[pl.ds(g * L, L)]
        plsc.addupdate_scatter(deg_l, [r16], ones)
        self_v = jnp.where(r16 == c16, 1.0, 0.0).astype(jnp.float32)
        plsc.addupdate_scatter(m_l, [r16], self_v)
        return _

    lax.fori_loop(0, EPW // L, edge_body, None)

    def km_body(g, _):
        k16 = kbuf[pl.ds(g * L, L)]
        plsc.addupdate_scatter(kf_l, [k16], ones)
        return _

    lax.fori_loop(0, KPW // L, km_body, None)

    pltpu.sync_copy(deg_l, part_hbm.at[pl.ds((w * 3 + 0) * NPAD, NPAD)])
    pltpu.sync_copy(kf_l, part_hbm.at[pl.ds((w * 3 + 1) * NPAD, NPAD)])
    pltpu.sync_copy(m_l, part_hbm.at[pl.ds((w * 3 + 2) * NPAD, NPAD)])


def _k2_body(part_hbm, x_hbm, dis_hbm, kn_hbm, m_hbm, y_hbm, ys_hbm,
             pbuf, dis_l, kn_l, m_acc, xbuf, ybuf, ysbuf):
    w = _wid()
    base = w * RPW
    for t in range(NW):
        for ch in range(3):
            pltpu.sync_copy(
                part_hbm.at[pl.ds((t * 3 + ch) * NPAD + base, RPW)],
                pbuf.at[pl.ds((t * 3 + ch) * RPW, RPW)])

    def red_body(g, _):
        deg16 = jnp.zeros((L,), jnp.float32)
        kf16 = jnp.zeros((L,), jnp.float32)
        m16 = jnp.zeros((L,), jnp.float32)
        for t in range(NW):
            deg16 = deg16 + pbuf[pl.ds((t * 3 + 0) * RPW + g * L, L)]
            kf16 = kf16 + pbuf[pl.ds((t * 3 + 1) * RPW + g * L, L)]
            m16 = m16 + pbuf[pl.ds((t * 3 + 2) * RPW + g * L, L)]
        pos = jnp.where(deg16 > 0.0, 1.0, 0.0).astype(jnp.float32)
        dis16 = _rsqrt16(jnp.maximum(deg16, 1.0)) * pos
        kn16 = jnp.where(kf16 > 0.0, 1.0, 0.0).astype(jnp.float32)
        dis_l[pl.ds(g * L, L)] = dis16
        kn_l[pl.ds(g * L, L)] = kn16
        m_acc[pl.ds(g * L, L)] = m16
        return _

    lax.fori_loop(0, RPW // L, red_body, None)

    pltpu.sync_copy(dis_l, dis_hbm.at[pl.ds(base, RPW)])
    pltpu.sync_copy(kn_l, kn_hbm.at[pl.ds(base, RPW)])
    pltpu.sync_copy(m_acc, m_hbm.at[pl.ds(base, RPW)])

    for b in range(RPW // RSUB):
        rb = base + b * RSUB
        pltpu.sync_copy(x_hbm.at[pl.ds(rb, RSUB), :], xbuf)

        def grp_body(g, _):
            kvec = kn_l[pl.ds(b * RSUB + g * L, L)]
            dvec = dis_l[pl.ds(b * RSUB + g * L, L)]
            for r in range(L):
                row = g * L + r
                s16 = jnp.zeros((L,), jnp.float32)
                xv = []
                for j in range(D // L):
                    v = xbuf[row, pl.ds(j * L, L)]
                    xv.append(v)
                    s16 = s16 + v * v
                tot = jnp.sum(s16)
                t16 = jnp.full((L,), tot, jnp.float32)
                inv16 = _rsqrt16(jnp.maximum(t16, 1e-24))
                kn16 = jnp.full((L,), kvec[r], jnp.float32)
                ds16 = jnp.full((L,), dvec[r], jnp.float32)
                for j in range(D // L):
                    yv = xv[j] * inv16 * kn16
                    ybuf[row, pl.ds(j * L, L)] = yv
                    ysbuf[row, pl.ds(j * L, L)] = yv * ds16
            return _

        lax.fori_loop(0, RSUB // L, grp_body, None)
        pltpu.sync_copy(ybuf, y_hbm.at[pl.ds(rb, RSUB), :])
        pltpu.sync_copy(ysbuf, ys_hbm.at[pl.ds(rb, RSUB), :])


def _k3_body(row_hbm, col_hbm, ys_hbm, part_hbm,
             acc, rbuf, cbuf, rx0, rx1, gb0, gb1, sem0, sem1):
    c = lax.axis_index("c")
    s = lax.axis_index("s")
    w = s * NC + c
    rx = (rx0, rx1)
    gb = (gb0, gb1)
    sems = (sem0, sem1)

    def zero_rows(r, _):
        for j in range(D // L):
            gb0[r, pl.ds(j * L, L)] = jnp.zeros((L,), jnp.float32)
        return _

    lax.fori_loop(0, ECH, zero_rows, None)
    for kk in range(RPS // ECH):
        pltpu.sync_copy(gb0, acc.at[pl.ds(s * RPS + kk * ECH, ECH), :])
    plsc.subcore_barrier()

    ebase = w * EPW
    pltpu.sync_copy(row_hbm.at[pl.ds(ebase, EPW)], rbuf)
    pltpu.sync_copy(col_hbm.at[pl.ds(ebase, EPW)], cbuf)

    def _fill_rx(dst, ch):
        for k in range(ECH // L):
            dst[pl.ds(k * L, L)] = rbuf[pl.ds(ch * ECH + k * L, L)]

    for b in range(2):
        _fill_rx(rx[b], b)
        pltpu.async_copy(ys_hbm.at[cbuf.at[pl.ds(b * ECH, ECH)]],
                         gb[b], sems[b])

    def ring_body(g2, _):
        for b in range(2):
            ch = g2 * 2 + b
            pltpu.make_async_copy(ys_hbm.at[cbuf.at[pl.ds(ch * ECH, ECH)]],
                                  gb[b], sems[b]).wait()
            pltpu.sync_copy(gb[b], acc.at[rx[b]], add=True)

            @pl.when(ch + 2 < NCH)
            def _issue():
                _fill_rx(rx[b], ch + 2)
                pltpu.async_copy(ys_hbm.at[cbuf.at[pl.ds((ch + 2) * ECH,
                                                         ECH)]],
                                 gb[b], sems[b])
        return _

    lax.fori_loop(0, NCH // 2, ring_body, None)
    last = NCH - 1
    pltpu.make_async_copy(ys_hbm.at[cbuf.at[pl.ds(last * ECH, ECH)]],
                          gb0, sem0).wait()
    pltpu.sync_copy(gb0, acc.at[rx0], add=True)

    plsc.subcore_barrier()
    for kk in range(RPS // ECH):
        rb = s * RPS + kk * ECH
        pltpu.sync_copy(acc.at[pl.ds(rb, ECH), :], gb0)
        pltpu.sync_copy(gb0, part_hbm.at[c, pl.ds(rb, ECH), :])


def _k4_body(part_hbm, dis_hbm, kn_hbm, m_hbm, y_hbm, ys_hbm, out_hbm,
             p0, p1, yb, ysb, ob, dis_l, kn_l, m_l):
    w = _wid()
    base = w * RPW
    pltpu.sync_copy(dis_hbm.at[pl.ds(base, RPW)], dis_l)
    pltpu.sync_copy(kn_hbm.at[pl.ds(base, RPW)], kn_l)
    pltpu.sync_copy(m_hbm.at[pl.ds(base, RPW)], m_l)

    for b in range(RPW // BSUB):
        rb = base + b * BSUB
        pltpu.sync_copy(part_hbm.at[0, pl.ds(rb, BSUB), :], p0)
        pltpu.sync_copy(part_hbm.at[1, pl.ds(rb, BSUB), :], p1)
        pltpu.sync_copy(y_hbm.at[pl.ds(rb, BSUB), :], yb)
        pltpu.sync_copy(ys_hbm.at[pl.ds(rb, BSUB), :], ysb)

        def grp_body(g, _):
            dvec = dis_l[pl.ds(b * BSUB + g * L, L)]
            kvec = kn_l[pl.ds(b * BSUB + g * L, L)]
            mvec = m_l[pl.ds(b * BSUB + g * L, L)]
            for r in range(L):
                row = g * L + r
                d16 = jnp.full((L,), dvec[r], jnp.float32)
                k16 = jnp.full((L,), kvec[r], jnp.float32)
                m16 = jnp.full((L,), mvec[r], jnp.float32)
                nk16 = 1.0 - k16
                for j in range(D // L):
                    sl = pl.ds(j * L, L)
                    prop = d16 * (p0[row, sl] + p1[row, sl]
                                  - m16 * ysb[row, sl])
                    ob[row, sl] = k16 * yb[row, sl] + nk16 * prop
            return _

        lax.fori_loop(0, BSUB // L, grp_body, None)
        pltpu.sync_copy(ob, out_hbm.at[pl.ds(rb, BSUB), :])


@functools.cache
def _build():
    mesh = plsc.VectorSubcoreMesh(
        core_axis_name="c", subcore_axis_name="s",
        num_cores=NC, num_subcores=NS)
    f32 = jnp.float32
    cparams = pltpu.CompilerParams(needs_layout_passes=False)

    k1 = pl.kernel(
        _k1_body,
        out_type=jax.ShapeDtypeStruct((NW * 3 * NPAD,), f32),
        mesh=mesh,
        compiler_params=cparams,
        scratch_types=[
            pltpu.VMEM((EPW,), jnp.int32),
            pltpu.VMEM((EPW,), jnp.int32),
            pltpu.VMEM((KPW,), jnp.int32),
            pltpu.VMEM((NPAD,), f32),
            pltpu.VMEM((NPAD,), f32),
            pltpu.VMEM((NPAD,), f32),
        ],
    )
    k2 = pl.kernel(
        _k2_body,
        out_type=[
            jax.ShapeDtypeStruct((NPAD,), f32),
            jax.ShapeDtypeStruct((NPAD,), f32),
            jax.ShapeDtypeStruct((NPAD,), f32),
            jax.ShapeDtypeStruct((NPAD, D), f32),
            jax.ShapeDtypeStruct((NPAD, D), f32),
        ],
        mesh=mesh,
        compiler_params=cparams,
        scratch_types=[
            pltpu.VMEM((NW * 3 * RPW,), f32),
            pltpu.VMEM((RPW,), f32),
            pltpu.VMEM((RPW,), f32),
            pltpu.VMEM((RPW,), f32),
            pltpu.VMEM((RSUB, D), f32),
            pltpu.VMEM((RSUB, D), f32),
            pltpu.VMEM((RSUB, D), f32),
        ],
    )
    k3 = pl.kernel(
        _k3_body,
        out_type=jax.ShapeDtypeStruct((NC, NPAD, D), f32),
        mesh=mesh,
        compiler_params=cparams,
        scratch_types=[
            pltpu.VMEM_SHARED((NPAD, D), f32),
            pltpu.VMEM((EPW,), jnp.int32),
            pltpu.VMEM((EPW,), jnp.int32),
            pltpu.VMEM((ECH,), jnp.int32),
            pltpu.VMEM((ECH,), jnp.int32),
            pltpu.VMEM((ECH, D), f32),
            pltpu.VMEM((ECH, D), f32),
            pltpu.SemaphoreType.DMA,
            pltpu.SemaphoreType.DMA,
        ],
    )
    k4 = pl.kernel(
        _k4_body,
        out_type=jax.ShapeDtypeStruct((NPAD, D), f32),
        mesh=mesh,
        compiler_params=cparams,
        scratch_types=[
            pltpu.VMEM((BSUB, D), f32),
            pltpu.VMEM((BSUB, D), f32),
            pltpu.VMEM((BSUB, D), f32),
            pltpu.VMEM((BSUB, D), f32),
            pltpu.VMEM((BSUB, D), f32),
            pltpu.VMEM((RPW,), f32),
            pltpu.VMEM((RPW,), f32),
            pltpu.VMEM((RPW,), f32),
        ],
    )
    return k1, k2, k3, k4


@jax.jit
def kernel(x, edge_index, known_mask):
    k1, k2, k3, k4 = _build()
    row = edge_index[0]
    col = edge_index[1]
    x_pad = jnp.concatenate(
        [x, jnp.zeros((NPAD - N, D), jnp.float32)], axis=0)
    km_pad = jnp.concatenate(
        [known_mask, jnp.full((KPAD - K,), N, jnp.int32)], axis=0)

    part = k1(row, col, km_pad)
    dis, kn, m, y, ys = k2(part, x_pad)
    pacc = k3(row, col, ys)
    out_pad = k4(pacc, dis, kn, m, y, ys)
    return out_pad[:N]
